# baseline (speedup 1.0000x reference)
import jax
import jax.numpy as jnp
from jax.experimental import pallas as pl
from jax.experimental.pallas import tpu as pltpu

HIDDEN = 1024
FMIN = 0.1
FMAX = 0.5

B = 128
F = 32768
LW = 128
BK = 8192
NSTEP = 2
NCH = BK // LW


def _fold_block(ref, blk_id, vm, vi):
    for c in range(NCH):
        v = ref[:, c * LW : (c + 1) * LW]
        chunk_id = blk_id * NCH + c
        gt = v > vm
        vi = jnp.where(gt, chunk_id, vi)
        vm = jnp.where(gt, v, vm)
    return vm, vi


def _psd_peak_body(xa_ref, xb_ref, out_ref, rmax, ridx):
    k = pl.program_id(0)

    @pl.when(k == 0)
    def _():
        rmax[...] = jnp.full((B, LW), -jnp.inf, jnp.float32)
        ridx[...] = jnp.zeros((B, LW), jnp.int32)

    vm, vi = rmax[...], ridx[...]
    vm, vi = _fold_block(xa_ref, k, vm, vi)
    vm, vi = _fold_block(xb_ref, k + NSTEP, vm, vi)
    rmax[...] = vm
    ridx[...] = vi

    @pl.when(k == NSTEP - 1)
    def _():
        m = rmax[...]
        col = ridx[...] * LW + jax.lax.broadcasted_iota(jnp.int32, (B, LW), 1)
        gmax = jnp.max(m, axis=1, keepdims=True)
        cand = jnp.where(m == gmax, col, F)
        peak = jnp.min(cand, axis=1, keepdims=True)

        idxf = peak.astype(jnp.float32)
        freq = FMIN + (FMAX - FMIN) * idxf / (F - 1)
        rr = freq * 60.0
        out_ref[...] = jnp.broadcast_to(rr, (B, HIDDEN))


_psd_peak = pl.pallas_call(
    _psd_peak_body,
    grid=(NSTEP,),
    in_specs=[
        pl.BlockSpec((B, BK), lambda k: (0, k)),
        pl.BlockSpec((B, BK), lambda k: (0, k + NSTEP)),
    ],
    out_specs=pl.BlockSpec((B, HIDDEN), lambda k: (0, 0)),
    out_shape=jax.ShapeDtypeStruct((B, HIDDEN), jnp.float32),
    scratch_shapes=[
        pltpu.VMEM((B, LW), jnp.float32),
        pltpu.VMEM((B, LW), jnp.int32),
    ],
)


def kernel(x):
    return _psd_peak(x, x)
